# baseline (speedup 1.0000x reference)
import jax
import jax.numpy as jnp
from jax.experimental import pallas as pl

N_PATCHES = 1024
MODEL_DIM = 768
P_BLK = 256


def _add_kernel(patches_ref, pos_ref, out_ref):
    out_ref[...] = patches_ref[...] + pos_ref[...]


def kernel(patches, pos_table):
    batch = patches.shape[0]
    grid = (batch, N_PATCHES // P_BLK)
    return pl.pallas_call(
        _add_kernel,
        grid=grid,
        in_specs=[
            pl.BlockSpec((1, P_BLK, MODEL_DIM), lambda b, j: (b, j, 0)),
            pl.BlockSpec((P_BLK, MODEL_DIM), lambda b, j: (j, 0)),
        ],
        out_specs=pl.BlockSpec((1, P_BLK, MODEL_DIM), lambda b, j: (b, j, 0)),
        out_shape=jax.ShapeDtypeStruct(patches.shape, patches.dtype),
    )(patches, pos_table)


# speedup vs baseline: 1.1585x; 1.1585x over previous
import jax
import jax.numpy as jnp
from jax.experimental import pallas as pl

N_PATCHES = 1024
MODEL_DIM = 768
P_BLK = 256


def _add_kernel(patches_ref, pos_ref, out_ref):
    out_ref[...] = patches_ref[...] + pos_ref[...]


def kernel(patches, pos_table):
    batch = patches.shape[0]
    grid = (N_PATCHES // P_BLK, batch)
    return pl.pallas_call(
        _add_kernel,
        grid=grid,
        in_specs=[
            pl.BlockSpec((1, P_BLK, MODEL_DIM), lambda j, b: (b, j, 0)),
            pl.BlockSpec((P_BLK, MODEL_DIM), lambda j, b: (j, 0)),
        ],
        out_specs=pl.BlockSpec((1, P_BLK, MODEL_DIM), lambda j, b: (b, j, 0)),
        out_shape=jax.ShapeDtypeStruct(patches.shape, patches.dtype),
    )(patches, pos_table)


# speedup vs baseline: 1.9984x; 1.7249x over previous
import jax
import jax.numpy as jnp
from jax.experimental import pallas as pl

N_PATCHES = 1024
MODEL_DIM = 768
P_BLK = 256


def _add_kernel(patches_ref, pos_ref, out_ref):
    out_ref[...] = patches_ref[...] + pos_ref[...]


def kernel(patches, pos_table):
    batch = patches.shape[0]
    flat = patches.reshape(batch * N_PATCHES, MODEL_DIM)
    out = pl.pallas_call(
        _add_kernel,
        grid=(batch,),
        in_specs=[
            pl.BlockSpec((N_PATCHES, MODEL_DIM), lambda b: (b, 0)),
            pl.BlockSpec((N_PATCHES, MODEL_DIM), lambda b: (0, 0)),
        ],
        out_specs=pl.BlockSpec((N_PATCHES, MODEL_DIM), lambda b: (b, 0)),
        out_shape=jax.ShapeDtypeStruct(flat.shape, flat.dtype),
    )(flat, pos_table)
    return out.reshape(patches.shape)
